# Initial kernel scaffold; baseline (speedup 1.0000x reference)
import jax
import jax.numpy as jnp
from jax.experimental import pallas as pl


def kernel(x, table1, table2):
    raise NotImplementedError("write your pallas kernel here")


# speedup vs baseline: 1.8930x; 1.8930x over previous
import functools

import jax
import jax.numpy as jnp
from jax import lax
from jax.experimental import pallas as pl
from jax.experimental.pallas import tpu as pltpu
from jax.experimental.pallas import tpu_sc as plsc

_NUM_EMB = 1000000
_HALF = 32
_BATCH = 16384
_FIELDS = 26
_BF = _BATCH * _FIELDS
_NC = 2
_NS = 16
_NW = _NC * _NS
_PER_W = _BF // _NW
_GRP = 128
_NG = _PER_W // _GRP
_BLK = 1024
_GPB = _BLK // _GRP
_NBLK = _PER_W // _BLK


def _dual_gather(x_grp, table1, table2):
    mesh = plsc.VectorSubcoreMesh(core_axis_name="c", subcore_axis_name="s")

    @functools.partial(
        pl.kernel,
        mesh=mesh,
        compiler_params=pltpu.CompilerParams(use_tc_tiling_on_sc=False),
        out_type=jax.ShapeDtypeStruct((_BF, 2 * _HALF), jnp.float32),
        scratch_types=[
            pltpu.VMEM((_NG, _GRP), jnp.int32),
            pltpu.VMEM((_BLK, _HALF), jnp.float32),
            pltpu.VMEM((_BLK, _HALF), jnp.float32),
            pltpu.SemaphoreType.DMA,
        ],
    )
    def k(x_hbm, t1_hbm, t2_hbm, out_hbm, idx_v, rows1_v, rows2_v, sem):
        wid = lax.axis_index("s") * _NC + lax.axis_index("c")
        base = wid * _PER_W
        pltpu.sync_copy(x_hbm.at[wid], idx_v)

        def blk_body(j, carry):
            handles = []
            for g in range(_GPB):
                row = j * _GPB + g
                handles.append(pltpu.async_copy(
                    t1_hbm.at[idx_v.at[row]],
                    rows1_v.at[pl.ds(g * _GRP, _GRP)], sem))
                handles.append(pltpu.async_copy(
                    t2_hbm.at[idx_v.at[row]],
                    rows2_v.at[pl.ds(g * _GRP, _GRP)], sem))
            for h in handles:
                h.wait()
            obase = base + j * _BLK
            pltpu.sync_copy(
                rows1_v, out_hbm.at[pl.ds(obase, _BLK), pl.ds(0, _HALF)])
            pltpu.sync_copy(
                rows2_v, out_hbm.at[pl.ds(obase, _BLK), pl.ds(_HALF, _HALF)])
            return carry

        lax.fori_loop(0, _NBLK, blk_body, 0)

    return k(x_grp, table1, table2)


def kernel(x, table1, table2):
    x_grp = x.reshape(_NW, _NG, _GRP).astype(jnp.int32)
    out = _dual_gather(x_grp, table1, table2)
    return out.reshape(_BATCH, _FIELDS, 2 * _HALF)


# speedup vs baseline: 1.8938x; 1.0005x over previous
import functools

import jax
import jax.numpy as jnp
from jax import lax
from jax.experimental import pallas as pl
from jax.experimental.pallas import tpu as pltpu
from jax.experimental.pallas import tpu_sc as plsc

_NUM_EMB = 1000000
_HALF = 32
_BATCH = 16384
_FIELDS = 26
_BF = _BATCH * _FIELDS
_NC = 2
_NS = 16
_NW = _NC * _NS
_PER_W = _BF // _NW
_GRP = 128
_NG = _PER_W // _GRP
_BLK = 512
_GPB = _BLK // _GRP
_NBLK = _PER_W // _BLK


def _dual_gather(x_grp, table1, table2):
    mesh = plsc.VectorSubcoreMesh(core_axis_name="c", subcore_axis_name="s")

    @functools.partial(
        pl.kernel,
        mesh=mesh,
        compiler_params=pltpu.CompilerParams(use_tc_tiling_on_sc=False),
        out_type=jax.ShapeDtypeStruct((_BF, 2 * _HALF), jnp.float32),
        scratch_types=[
            pltpu.VMEM((_NG, _GRP), jnp.int32),
            pltpu.VMEM((2, _BLK, _HALF), jnp.float32),
            pltpu.VMEM((2, _BLK, _HALF), jnp.float32),
            pltpu.SemaphoreType.DMA,
            pltpu.SemaphoreType.DMA,
        ],
    )
    def k(x_hbm, t1_hbm, t2_hbm, out_hbm, idx_v, rows1_v, rows2_v,
          gsem, wsem):
        wid = lax.axis_index("s") * _NC + lax.axis_index("c")
        base = wid * _PER_W
        pltpu.sync_copy(x_hbm.at[wid], idx_v)

        def outer(i, carry):
            for b in range(2):
                j = 2 * i + b
                ghs = []
                for g in range(_GPB):
                    row = j * _GPB + g
                    ghs.append(pltpu.async_copy(
                        t1_hbm.at[idx_v.at[row]],
                        rows1_v.at[b].at[pl.ds(g * _GRP, _GRP)], gsem))
                    ghs.append(pltpu.async_copy(
                        t2_hbm.at[idx_v.at[row]],
                        rows2_v.at[b].at[pl.ds(g * _GRP, _GRP)], gsem))

                pb = 1 - b
                pbase = base + (j - 1) * _BLK

                @pl.when(j > 0)
                def _fire_writes():
                    pltpu.async_copy(
                        rows1_v.at[pb],
                        out_hbm.at[pl.ds(pbase, _BLK), pl.ds(0, _HALF)],
                        wsem)
                    pltpu.async_copy(
                        rows2_v.at[pb],
                        out_hbm.at[pl.ds(pbase, _BLK), pl.ds(_HALF, _HALF)],
                        wsem)

                for h in ghs:
                    h.wait()

                @pl.when(j > 0)
                def _wait_writes():
                    pltpu.make_async_copy(
                        rows1_v.at[pb],
                        out_hbm.at[pl.ds(pbase, _BLK), pl.ds(0, _HALF)],
                        wsem).wait()
                    pltpu.make_async_copy(
                        rows2_v.at[pb],
                        out_hbm.at[pl.ds(pbase, _BLK), pl.ds(_HALF, _HALF)],
                        wsem).wait()
            return carry

        lax.fori_loop(0, _NBLK // 2, outer, 0)

        lbase = base + (_NBLK - 1) * _BLK
        pltpu.sync_copy(
            rows1_v.at[1], out_hbm.at[pl.ds(lbase, _BLK), pl.ds(0, _HALF)])
        pltpu.sync_copy(
            rows2_v.at[1],
            out_hbm.at[pl.ds(lbase, _BLK), pl.ds(_HALF, _HALF)])

    return k(x_grp, table1, table2)


def kernel(x, table1, table2):
    x_grp = x.reshape(_NW, _NG, _GRP).astype(jnp.int32)
    out = _dual_gather(x_grp, table1, table2)
    return out.reshape(_BATCH, _FIELDS, 2 * _HALF)
